# baseline (speedup 1.0000x reference)
import jax
import jax.numpy as jnp
from jax import lax
from jax.experimental import pallas as pl
from jax.experimental.pallas import tpu as pltpu

TOPK_KS = (1, 5, 20)


def _make_tc_gather(B, V, P=8, CB=128):
    G = B // P

    def body(tgt_smem, *refs):
        xs, o_ref = refs[:P], refs[P]
        i = pl.program_id(0)
        lane = lax.broadcasted_iota(jnp.int32, (1, CB), 1)
        rowid = lax.broadcasted_iota(jnp.int32, (P, 1), 0)
        res = jnp.zeros((P, 1), jnp.float32)
        for j in range(P):
            tg = tgt_smem[i * P + j]
            sel = jnp.where(lane == tg % CB, xs[j][j:j + 1, :], 0.0)
            res = jnp.where(rowid == j, jnp.sum(sel), res)
        o_ref[...] = res

    def imap(j):
        return lambda i, tgt: (i, tgt[i * P + j] // CB)

    return pl.pallas_call(
        body,
        grid_spec=pltpu.PrefetchScalarGridSpec(
            num_scalar_prefetch=1,
            grid=(G,),
            in_specs=[pl.BlockSpec((P, CB), imap(j)) for j in range(P)],
            out_specs=pl.BlockSpec((P, 1), lambda i, tgt: (i, 0)),
        ),
        out_shape=jax.ShapeDtypeStruct((B, 1), jnp.float32),
    )


def _make_count(B, V, BC):
    ncb = -(-V // BC)
    scale = 100.0 / B

    def count_kernel(t_ref, tgt_ref, x_ref, o_ref, acc_ref):
        i = pl.program_id(0)

        @pl.when(i == 0)
        def _init():
            acc_ref[...] = jnp.zeros_like(acc_ref)

        x = x_ref[...]
        t = t_ref[...]
        tg = tgt_ref[...]
        col = lax.broadcasted_iota(jnp.int32, (B, BC), 1) + i * BC
        before = (x > t) | ((x == t) & (col < tg))
        before &= col < V
        acc_ref[...] += jnp.sum(before.astype(jnp.int32), axis=1,
                                keepdims=True)

        @pl.when(i == ncb - 1)
        def _fin():
            rank = acc_ref[...]
            sums = [jnp.sum((rank < k).astype(jnp.float32)) * scale
                    for k in TOPK_KS]
            rowid = lax.broadcasted_iota(jnp.int32, (8, 128), 0)
            res = jnp.zeros((8, 128), jnp.float32)
            for j, s in enumerate(sums):
                res = jnp.where(rowid == j, s, res)
            o_ref[...] = res

    return pl.pallas_call(
        count_kernel,
        grid=(ncb,),
        in_specs=[
            pl.BlockSpec((B, 1), lambda i: (0, 0)),
            pl.BlockSpec((B, 1), lambda i: (0, 0)),
            pl.BlockSpec((B, BC), lambda i: (0, i)),
        ],
        out_specs=pl.BlockSpec((8, 128), lambda i: (0, 0)),
        out_shape=jax.ShapeDtypeStruct((8, 128), jnp.float32),
        scratch_shapes=[pltpu.VMEM((B, 1), jnp.int32)],
    )


def kernel(outputs, targets):
    B, V = outputs.shape
    targets = targets.astype(jnp.int32)
    t = _make_tc_gather(B, V, P=16)(targets, *([outputs] * 16))
    out = _make_count(B, V, 4096)(t, targets.reshape(B, 1), outputs)
    return out[:3, :1]
